# baseline (speedup 1.0000x reference)
import functools

import jax
import jax.numpy as jnp
from jax import lax
from jax.experimental import pallas as pl
from jax.experimental.pallas import tpu as pltpu
from jax.experimental.pallas import tpu_sc as plsc

_EPS_BN = 1e-5
_BM_EXP = 128


def _mlp_body(*refs, preact, stats):
    if preact:
        x_ref, w_ref, b_ref, s_ref, t_ref = refs[:5]
        refs = refs[5:]
    else:
        x_ref, w_ref, b_ref = refs[:3]
        refs = refs[3:]
    y_ref = refs[0]
    i = pl.program_id(0)
    x = x_ref[...]
    if preact:
        x = jnp.maximum(x * s_ref[...] + t_ref[...], 0.0)
    y = jnp.dot(x.astype(jnp.bfloat16), w_ref[...].astype(jnp.bfloat16),
                preferred_element_type=jnp.float32) + b_ref[...]
    y_ref[...] = y
    if stats:
        stats_ref = refs[1]

        @pl.when(i == 0)
        def _():
            stats_ref[...] = jnp.zeros_like(stats_ref)

        stats_ref[0:1, :] += jnp.sum(y, axis=0, keepdims=True)
        stats_ref[1:2, :] += jnp.sum(y * y, axis=0, keepdims=True)


def _mlp(x, w, b, s, t, *, preact=True, stats=True):
    m, k = x.shape
    n = w.shape[1]
    bm = min(512, m)
    grid = (m // bm,)
    in_specs = [
        pl.BlockSpec((bm, k), lambda i: (i, 0)),
        pl.BlockSpec((k, n), lambda i: (0, 0)),
        pl.BlockSpec((1, n), lambda i: (0, 0)),
    ]
    args = [x, w, b.reshape(1, n)]
    if preact:
        in_specs += [pl.BlockSpec((1, k), lambda i: (0, 0)),
                     pl.BlockSpec((1, k), lambda i: (0, 0))]
        args += [s.reshape(1, k), t.reshape(1, k)]
    out_shape = [jax.ShapeDtypeStruct((m, n), jnp.float32)]
    out_specs = [pl.BlockSpec((bm, n), lambda i: (i, 0))]
    if stats:
        out_shape.append(jax.ShapeDtypeStruct((8, n), jnp.float32))
        out_specs.append(pl.BlockSpec((8, n), lambda i: (0, 0)))
    body = functools.partial(_mlp_body, preact=preact, stats=stats)
    res = pl.pallas_call(
        body, grid=grid, in_specs=in_specs, out_specs=out_specs,
        out_shape=out_shape)(*args)
    return res if stats else res[0]


def _expert_body(h_ref, w1_ref, c_ref, o_ref, st_ref, *, e_total):
    i = pl.program_id(0)
    j = pl.program_id(1)
    h = h_ref[...]
    eo = jnp.maximum(
        jnp.dot(h.astype(jnp.bfloat16), w1_ref[0].astype(jnp.bfloat16),
                preferred_element_type=jnp.float32), 0.0)
    lane = jax.lax.broadcasted_iota(jnp.int32, (1, c_ref.shape[1]), 1)
    sel = jnp.sum(c_ref[...] * (lane == j).astype(jnp.float32), axis=1,
                  keepdims=True)

    @pl.when(j == 0)
    def _():
        o_ref[...] = h

    o_ref[...] += sel * eo

    @pl.when(j == e_total - 1)
    def _():
        @pl.when(i == 0)
        def _():
            st_ref[...] = jnp.zeros_like(st_ref)

        o = o_ref[...]
        st_ref[0:1, :] += jnp.sum(o, axis=0, keepdims=True)
        st_ref[1:2, :] += jnp.sum(o * o, axis=0, keepdims=True)


def _expert_stats(h, w1, combine):
    m, n = h.shape
    e = w1.shape[0]
    bm = min(1024, m)
    body = functools.partial(_expert_body, e_total=e)
    return pl.pallas_call(
        body, grid=(m // bm, e),
        in_specs=[
            pl.BlockSpec((bm, n), lambda i, j: (i, 0)),
            pl.BlockSpec((1, n, n), lambda i, j: (j, 0, 0)),
            pl.BlockSpec((bm, e), lambda i, j: (i, 0)),
        ],
        out_specs=[
            pl.BlockSpec((bm, n), lambda i, j: (i, 0)),
            pl.BlockSpec((8, n), lambda i, j: (0, 0)),
        ],
        out_shape=[
            jax.ShapeDtypeStruct((m, n), jnp.float32),
            jax.ShapeDtypeStruct((8, n), jnp.float32),
        ])(h, w1, combine)


def _sc_gather(table, idx):
    v, d = table.shape
    b = idx.shape[0]
    info = plsc.get_sparse_core_info()
    nw = info.num_cores * info.num_subcores
    bpw = b // nw
    ch = 32
    nch = bpw // ch
    mesh = plsc.VectorSubcoreMesh(core_axis_name="c", subcore_axis_name="s")

    @functools.partial(
        pl.kernel, mesh=mesh,
        out_type=jax.ShapeDtypeStruct((b, d), jnp.float32),
        scratch_types=[
            pltpu.VMEM((bpw,), jnp.int32),
            pltpu.VMEM((ch, d), jnp.float32),
            pltpu.SemaphoreType.DMA,
        ])
    def k(table_hbm, idx_hbm, out_hbm, idx_v, rows_v, sem):
        wid = lax.axis_index("s") * info.num_cores + lax.axis_index("c")
        base = wid * bpw
        pltpu.sync_copy(idx_hbm.at[pl.ds(base, bpw)], idx_v)

        def body(c, carry):
            pltpu.async_copy(
                table_hbm.at[idx_v.at[pl.ds(c * ch, ch)]], rows_v, sem).wait()
            pltpu.sync_copy(rows_v, out_hbm.at[pl.ds(base + c * ch, ch)])
            return carry

        lax.fori_loop(0, nch, body, 0)

    return k(table, idx)


def _grouped_body(be_ref, h_ref, w_ref, ws_ref, o_ref):
    eo = jnp.maximum(
        jnp.dot(h_ref[...].astype(jnp.bfloat16),
                w_ref[0].astype(jnp.bfloat16),
                preferred_element_type=jnp.float32), 0.0)
    o_ref[...] = ws_ref[...] * eo


def _grouped_mm(h_sorted, w1, wsort, block_e):
    t, n = h_sorted.shape
    nb = block_e.shape[0]
    bm = t // nb
    gs = pltpu.PrefetchScalarGridSpec(
        num_scalar_prefetch=1, grid=(nb,),
        in_specs=[
            pl.BlockSpec((bm, n), lambda i, be: (i, 0)),
            pl.BlockSpec((1, n, n), lambda i, be: (be[i], 0, 0)),
            pl.BlockSpec((bm, 1), lambda i, be: (i, 0)),
        ],
        out_specs=pl.BlockSpec((bm, n), lambda i, be: (i, 0)))
    return pl.pallas_call(
        _grouped_body, grid_spec=gs,
        out_shape=jax.ShapeDtypeStruct((t, n), jnp.float32),
    )(block_e, h_sorted, w1, wsort.reshape(t, 1))


def _combine_body(h_ref, ea_ref, eb_ref, o_ref, st_ref):
    i = pl.program_id(0)
    o = h_ref[...] + (ea_ref[...] + eb_ref[...])
    o_ref[...] = o

    @pl.when(i == 0)
    def _():
        st_ref[...] = jnp.zeros_like(st_ref)

    st_ref[0:1, :] += jnp.sum(o, axis=0, keepdims=True)
    st_ref[1:2, :] += jnp.sum(o * o, axis=0, keepdims=True)


def _combine_stats(hh, eo2):
    m, n = hh.shape
    bm = 512
    nblk = m // bm
    return pl.pallas_call(
        _combine_body, grid=(nblk,),
        in_specs=[
            pl.BlockSpec((bm, n), lambda i: (i, 0)),
            pl.BlockSpec((bm, n), lambda i: (i, 0)),
            pl.BlockSpec((bm, n), lambda i, _n=nblk: (i + _n, 0)),
        ],
        out_specs=[
            pl.BlockSpec((bm, n), lambda i: (i, 0)),
            pl.BlockSpec((8, n), lambda i: (0, 0)),
        ],
        out_shape=[
            jax.ShapeDtypeStruct((m, n), jnp.float32),
            jax.ShapeDtypeStruct((8, n), jnp.float32),
        ])(hh, eo2, eo2)


def _routed_expert_stats(hh, w1, top2_idx, top2_w):
    m, n = hh.shape
    e = w1.shape[0]
    bm = _BM_EXP
    e1, e2 = top2_idx[:, 0], top2_idx[:, 1]
    w1_, w2_ = top2_w[:, 0], top2_w[:, 1]
    swap = e2 < e1
    ea = jnp.where(swap, e2, e1)
    eb = jnp.where(swap, e1, e2)
    wa = jnp.where(swap, w2_, w1_)
    wb = jnp.where(swap, w1_, w2_)
    oh_a = jax.nn.one_hot(ea, e, dtype=jnp.int32)
    oh_b = jax.nn.one_hot(eb, e, dtype=jnp.int32)
    occ = oh_a + oh_b
    excl = jnp.cumsum(occ, axis=0) - occ
    rank_a = jnp.take_along_axis(excl, ea[:, None], axis=1)[:, 0]
    rank_b = jnp.take_along_axis(excl, eb[:, None], axis=1)[:, 0]
    counts = jnp.sum(occ, axis=0)
    padded = ((counts + bm - 1) // bm) * bm
    off = jnp.concatenate([jnp.zeros((1,), jnp.int32),
                           jnp.cumsum(padded).astype(jnp.int32)])
    da = off[ea] + rank_a
    db = off[eb] + rank_b
    t_pad = 2 * m + e * bm
    tok = jnp.arange(m, dtype=jnp.int32)
    disp = jnp.zeros((t_pad,), jnp.int32).at[da].set(tok).at[db].set(tok)
    wsort = jnp.zeros((t_pad,), jnp.float32).at[da].set(wa).at[db].set(wb)
    nb = t_pad // bm
    block_e = jnp.clip(
        jnp.searchsorted(off[1:], jnp.arange(nb, dtype=jnp.int32) * bm,
                         side='right'), 0, e - 1).astype(jnp.int32)
    h_sorted = _sc_gather(hh, disp)
    eo = _grouped_mm(h_sorted, w1, wsort, block_e)
    eo2 = _sc_gather(eo, jnp.concatenate([da, db]).astype(jnp.int32))
    return _combine_stats(hh, eo2)


def _affine(stats, g, beta, m):
    su, sq = stats[0], stats[1]
    mu = su / m
    var = sq / m - mu * mu
    s = g / jnp.sqrt(var + _EPS_BN)
    t = beta - mu * s
    return s, t


def _bn_expr(x, g, b):
    mu = jnp.mean(x, axis=0)
    var = jnp.var(x, axis=0)
    return g * (x - mu) / jnp.sqrt(var + _EPS_BN) + b


def _router(h, sp, m):
    hh = h @ sp['W'] + sp['b']
    logits = hh @ sp['gate']
    e = sp['gate'].shape[1]
    gates = jax.nn.softmax(logits, axis=-1)
    top2_vals, top2_idx = jax.lax.top_k(gates, 2)
    denom = jnp.sum(top2_vals, axis=-1, keepdims=True) + 1e-9
    top2_w = top2_vals / denom
    combine = jnp.zeros_like(gates).at[
        jnp.arange(m)[:, None], top2_idx].set(top2_w)
    one_hot_top1 = jax.nn.one_hot(top2_idx[:, 0], e, dtype=jnp.float32)
    density = jnp.mean(one_hot_top1, axis=0)
    density_proxy = jnp.mean(gates, axis=0)
    aux = jnp.mean(density * density_proxy) * float(e * e)
    return hh, combine, aux, top2_idx, top2_w


def kernel(x, params):
    p = params
    m = x.shape[0]
    h = x @ p['in_W'] + p['in_b']
    h = jax.nn.relu(_bn_expr(h, p['in_g'], p['in_beta']))
    loss = jnp.float32(0.0)
    for sp in p['sparse'][:-1]:
        hh, combine, aux, _, _ = _router(h, sp, m)
        loss = loss + aux
        expert_out = jax.nn.relu(jnp.einsum('nd,edh->enh', hh, sp['w1']))
        moe_out = jnp.einsum('ne,enh->nh', combine, expert_out)
        out = hh + moe_out
        h = jax.nn.relu(_bn_expr(out, sp['g'], sp['beta']))
    sp = p['sparse'][-1]
    hh, combine, aux, t2i, t2w = _router(h, sp, m)
    loss = loss + aux
    y, st = _routed_expert_stats(hh, sp['w1'], t2i, t2w)
    s, t = _affine(st, sp['g'], sp['beta'], m)
    for dp in p['dense']:
        y, st = _mlp(y, dp['W'], dp['b'], s, t)
        s, t = _affine(st, dp['g'], dp['beta'], m)
    out = _mlp(y, p['out_W'], p['out_b'], s, t, preact=True, stats=False)
    return out, loss


# speedup vs baseline: 1.0871x; 1.0871x over previous
import functools

import jax
import jax.numpy as jnp
from jax.experimental import pallas as pl

_EPS_BN = 1e-5


def _mlp_body(*refs, preact, stats):
    if preact:
        x_ref, w_ref, b_ref, s_ref, t_ref = refs[:5]
        refs = refs[5:]
    else:
        x_ref, w_ref, b_ref = refs[:3]
        refs = refs[3:]
    y_ref = refs[0]
    i = pl.program_id(0)
    x = x_ref[...]
    if preact:
        x = jnp.maximum(x * s_ref[...] + t_ref[...], 0.0)
    y = jnp.dot(x.astype(jnp.bfloat16), w_ref[...].astype(jnp.bfloat16),
                preferred_element_type=jnp.float32) + b_ref[...]
    y_ref[...] = y
    if stats:
        stats_ref = refs[1]

        @pl.when(i == 0)
        def _():
            stats_ref[...] = jnp.zeros_like(stats_ref)

        stats_ref[0:1, :] += jnp.sum(y, axis=0, keepdims=True)
        stats_ref[1:2, :] += jnp.sum(y * y, axis=0, keepdims=True)


def _mlp(x, w, b, s, t, *, preact=True, stats=True):
    m, k = x.shape
    n = w.shape[1]
    bm = min(512, m)
    grid = (m // bm,)
    in_specs = [
        pl.BlockSpec((bm, k), lambda i: (i, 0)),
        pl.BlockSpec((k, n), lambda i: (0, 0)),
        pl.BlockSpec((1, n), lambda i: (0, 0)),
    ]
    args = [x, w, b.reshape(1, n)]
    if preact:
        in_specs += [pl.BlockSpec((1, k), lambda i: (0, 0)),
                     pl.BlockSpec((1, k), lambda i: (0, 0))]
        args += [s.reshape(1, k), t.reshape(1, k)]
    out_shape = [jax.ShapeDtypeStruct((m, n), jnp.float32)]
    out_specs = [pl.BlockSpec((bm, n), lambda i: (i, 0))]
    if stats:
        out_shape.append(jax.ShapeDtypeStruct((8, n), jnp.float32))
        out_specs.append(pl.BlockSpec((8, n), lambda i: (0, 0)))
    body = functools.partial(_mlp_body, preact=preact, stats=stats)
    res = pl.pallas_call(
        body, grid=grid, in_specs=in_specs, out_specs=out_specs,
        out_shape=out_shape)(*args)
    return res if stats else res[0]


def _expert_body(h_ref, w1_ref, c_ref, o_ref, st_ref, *, e_total):
    i = pl.program_id(0)
    j = pl.program_id(1)
    h = h_ref[...]
    eo = jnp.maximum(
        jnp.dot(h.astype(jnp.bfloat16), w1_ref[0].astype(jnp.bfloat16),
                preferred_element_type=jnp.float32), 0.0)
    lane = jax.lax.broadcasted_iota(jnp.int32, (1, c_ref.shape[1]), 1)
    sel = jnp.sum(c_ref[...] * (lane == j).astype(jnp.float32), axis=1,
                  keepdims=True)

    @pl.when(j == 0)
    def _():
        o_ref[...] = h

    o_ref[...] += sel * eo

    @pl.when(j == e_total - 1)
    def _():
        @pl.when(i == 0)
        def _():
            st_ref[...] = jnp.zeros_like(st_ref)

        o = o_ref[...]
        st_ref[0:1, :] += jnp.sum(o, axis=0, keepdims=True)
        st_ref[1:2, :] += jnp.sum(o * o, axis=0, keepdims=True)


def _expert_stats(h, w1, combine):
    m, n = h.shape
    e = w1.shape[0]
    bm = min(1024, m)
    body = functools.partial(_expert_body, e_total=e)
    return pl.pallas_call(
        body, grid=(m // bm, e),
        in_specs=[
            pl.BlockSpec((bm, n), lambda i, j: (i, 0)),
            pl.BlockSpec((1, n, n), lambda i, j: (j, 0, 0)),
            pl.BlockSpec((bm, e), lambda i, j: (i, 0)),
        ],
        out_specs=[
            pl.BlockSpec((bm, n), lambda i, j: (i, 0)),
            pl.BlockSpec((8, n), lambda i, j: (0, 0)),
        ],
        out_shape=[
            jax.ShapeDtypeStruct((m, n), jnp.float32),
            jax.ShapeDtypeStruct((8, n), jnp.float32),
        ])(h, w1, combine)


def _affine(stats, g, beta, m):
    su, sq = stats[0], stats[1]
    mu = su / m
    var = sq / m - mu * mu
    s = g / jnp.sqrt(var + _EPS_BN)
    t = beta - mu * s
    return s, t


def _bn_expr(x, g, b):
    mu = jnp.mean(x, axis=0)
    var = jnp.var(x, axis=0)
    return g * (x - mu) / jnp.sqrt(var + _EPS_BN) + b


def _router(h, sp, m):
    hh = h @ sp['W'] + sp['b']
    logits = hh @ sp['gate']
    e = sp['gate'].shape[1]
    gates = jax.nn.softmax(logits, axis=-1)
    top2_vals, top2_idx = jax.lax.top_k(gates, 2)
    denom = jnp.sum(top2_vals, axis=-1, keepdims=True) + 1e-9
    top2_w = top2_vals / denom
    combine = jnp.zeros_like(gates).at[
        jnp.arange(m)[:, None], top2_idx].set(top2_w)
    one_hot_top1 = jax.nn.one_hot(top2_idx[:, 0], e, dtype=jnp.float32)
    density = jnp.mean(one_hot_top1, axis=0)
    density_proxy = jnp.mean(gates, axis=0)
    aux = jnp.mean(density * density_proxy) * float(e * e)
    return hh, combine, aux


def kernel(x, params):
    p = params
    m = x.shape[0]
    h = x @ p['in_W'] + p['in_b']
    h = jax.nn.relu(_bn_expr(h, p['in_g'], p['in_beta']))
    loss = jnp.float32(0.0)
    for sp in p['sparse'][:-1]:
        hh, combine, aux = _router(h, sp, m)
        loss = loss + aux
        expert_out = jax.nn.relu(jnp.einsum('nd,edh->enh', hh, sp['w1']))
        moe_out = jnp.einsum('ne,enh->nh', combine, expert_out)
        out = hh + moe_out
        h = jax.nn.relu(_bn_expr(out, sp['g'], sp['beta']))
    sp = p['sparse'][-1]
    hh, combine, aux = _router(h, sp, m)
    loss = loss + aux
    y, st = _expert_stats(hh, sp['w1'], combine)
    s, t = _affine(st, sp['g'], sp['beta'], m)
    for dp in p['dense']:
        y, st = _mlp(y, dp['W'], dp['b'], s, t)
        s, t = _affine(st, dp['g'], dp['beta'], m)
    out = _mlp(y, p['out_W'], p['out_b'], s, t, preact=True, stats=False)
    return out, loss
